# baseline (speedup 1.0000x reference)
import functools

import jax
import jax.numpy as jnp
from jax import lax
from jax.experimental import pallas as pl
from jax.experimental.pallas import tpu as pltpu
from jax.experimental.pallas import tpu_sc as plsc

BATCH = 16384
EMB = 16

_NC = 2
_NS = 16
_NW = _NC * _NS
_BPW = BATCH // _NW


def _gather_body(uid_hbm, iid_hbm, emb_u_hbm, emb_i_hbm, eu_hbm, ei_hbm,
                 uidx_v, iidx_v, rows_u, rows_i, sem_u, sem_i):
    wid = lax.axis_index("s") * _NC + lax.axis_index("c")
    base = wid * _BPW
    pltpu.sync_copy(uid_hbm.at[pl.ds(base, _BPW)], uidx_v)
    pltpu.sync_copy(iid_hbm.at[pl.ds(base, _BPW)], iidx_v)
    cu = pltpu.async_copy(emb_u_hbm.at[uidx_v], rows_u, sem_u)
    ci = pltpu.async_copy(emb_i_hbm.at[iidx_v], rows_i, sem_i)
    cu.wait()
    pltpu.sync_copy(rows_u, eu_hbm.at[pl.ds(base, _BPW)])
    ci.wait()
    pltpu.sync_copy(rows_i, ei_hbm.at[pl.ds(base, _BPW)])


@jax.jit
def _sc_gather(user_id, item_id, emb_user, emb_item):
    mesh = plsc.VectorSubcoreMesh(core_axis_name="c", subcore_axis_name="s")
    f = functools.partial(
        pl.kernel,
        mesh=mesh,
        compiler_params=pltpu.CompilerParams(use_tc_tiling_on_sc=False),
        out_type=(
            jax.ShapeDtypeStruct((BATCH, EMB), jnp.float32),
            jax.ShapeDtypeStruct((BATCH, EMB), jnp.float32),
        ),
        scratch_types=[
            pltpu.VMEM((_BPW,), jnp.int32),
            pltpu.VMEM((_BPW,), jnp.int32),
            pltpu.VMEM((_BPW, EMB), jnp.float32),
            pltpu.VMEM((_BPW, EMB), jnp.float32),
            pltpu.SemaphoreType.DMA,
            pltpu.SemaphoreType.DMA,
        ],
    )(_gather_body)
    return f(user_id, item_id, emb_user, emb_item)


def _mlp_body(eu_ref, ei_ref, w1a_ref, w1b_ref, b1_ref, w2_ref, b2_ref,
              w3_ref, b3_ref, out_ref):
    h = (jnp.dot(eu_ref[...], w1a_ref[...], preferred_element_type=jnp.float32)
         + jnp.dot(ei_ref[...], w1b_ref[...], preferred_element_type=jnp.float32)
         + b1_ref[...])
    h = jnp.maximum(h, 0.0)
    h2 = jnp.dot(h, w2_ref[...], preferred_element_type=jnp.float32) + b2_ref[...]
    h2 = jnp.maximum(h2, 0.0)
    o = jnp.dot(h2, w3_ref[...], preferred_element_type=jnp.float32) + b3_ref[...]
    out_ref[...] = jax.nn.sigmoid(o)


_MLP_BS = 2048


@jax.jit
def _tc_mlp(eu, ei, W1, b1, W2, b2, W3, b3):
    w1a = W1[:EMB]
    w1b = W1[EMB:]
    b1r = b1.reshape(1, -1)
    b2r = b2.reshape(1, -1)
    b3r = b3.reshape(1, -1)
    nblk = BATCH // _MLP_BS
    full = lambda i: (0, 0)
    return pl.pallas_call(
        _mlp_body,
        grid=(nblk,),
        in_specs=[
            pl.BlockSpec((_MLP_BS, EMB), lambda i: (i, 0)),
            pl.BlockSpec((_MLP_BS, EMB), lambda i: (i, 0)),
            pl.BlockSpec(w1a.shape, full),
            pl.BlockSpec(w1b.shape, full),
            pl.BlockSpec(b1r.shape, full),
            pl.BlockSpec(W2.shape, full),
            pl.BlockSpec(b2r.shape, full),
            pl.BlockSpec(W3.shape, full),
            pl.BlockSpec(b3r.shape, full),
        ],
        out_specs=pl.BlockSpec((_MLP_BS, 1), lambda i: (i, 0)),
        out_shape=jax.ShapeDtypeStruct((BATCH, 1), jnp.float32),
    )(eu, ei, w1a, w1b, b1r, W2, b2r, W3, b3r)


def kernel(input, emb_user, emb_item, W1, b1, W2, b2, W3, b3):
    user_id = input[:, 0].astype(jnp.int32)
    item_id = input[:, 1].astype(jnp.int32)
    rating = input[:, 2].astype(jnp.float32)
    eu, ei = _sc_gather(user_id, item_id, emb_user, emb_item)
    mlp_output = _tc_mlp(eu, ei, W1, b1, W2, b2, W3, b3)
    return (rating, mlp_output)
